# baseline (device time: 24843 ns/iter reference)
import numpy as np
import jax
import jax.numpy as jnp
from jax import lax
from jax.experimental import pallas as pl
from jax.experimental.pallas import tpu as pltpu

N_DEV = 4
B, Sq, D = 2, 256, 768
HQ_LOCAL, Dh = 4, 64
HD = HQ_LOCAL * Dh
CW = D // 2

F32 = jnp.float32
BF16 = jnp.bfloat16


def _rope_consts():
    inv = 1.0 / (10000.0 ** (np.arange(0, Dh, 2) / Dh))
    pos = np.arange(Sq)[:, None] * inv[None, :]
    cos = np.repeat(np.cos(pos), 2, axis=-1).astype(np.float32)
    sin = np.repeat(np.sin(pos), 2, axis=-1).astype(np.float32)
    rot = np.zeros((Dh, Dh), np.float32)
    for i in range(Dh // 2):
        rot[2 * i + 1, 2 * i] = -1.0
        rot[2 * i, 2 * i + 1] = 1.0
    cos4 = np.tile(cos, (1, HQ_LOCAL))
    sin4 = np.tile(sin, (1, HQ_LOCAL))
    rot4 = np.kron(np.eye(HQ_LOCAL, dtype=np.float32), rot)
    return jnp.asarray(cos4), jnp.asarray(sin4), jnp.asarray(rot4, BF16)


def kernel(x, Wq, Wk, Wv, Wo):
    cos4, sin4, rot4 = _rope_consts()

    def body(x_hbm, wq_hbm, wk_hbm, wv_hbm, wo_hbm, cos_hbm, sin_hbm,
             rot_hbm, out_hbm,
             xv, wqv, wkv, wvv, wov, cosv, sinv, rotv, acc_ref, recv_ref,
             load_sems, store_sems, send_sems, recv_sems):
        my = lax.axis_index("i")
        pa = 3 - my
        pb = my ^ 1

        pairs = [(x_hbm, xv), (wq_hbm, wqv), (wk_hbm, wkv), (wv_hbm, wvv),
                 (wo_hbm, wov), (cos_hbm, cosv), (sin_hbm, sinv),
                 (rot_hbm, rotv)]
        loads = []
        for i, (src, dst) in enumerate(pairs):
            cp = pltpu.make_async_copy(src, dst, load_sems.at[i])
            cp.start()
            loads.append(cp)

        barrier_sem = pltpu.get_barrier_semaphore()
        for nbr in [pa, pb]:
            pl.semaphore_signal(
                barrier_sem, inc=1,
                device_id=(nbr,), device_id_type=pl.DeviceIdType.MESH,
            )
        pl.semaphore_wait(barrier_sem, 2)
        for cp in loads:
            cp.wait()

        cos, sin = cosv[...], sinv[...]
        rot = rotv[...]
        wq16 = wqv[...].astype(BF16)
        wk16 = wkv[...].astype(BF16)
        wv16 = wvv[...].astype(BF16)
        wo16 = wov[...].astype(BF16)

        def exchange(step, wing, b, partner):
            idx = step * 4 + wing * 2 + b
            rdma = pltpu.make_async_remote_copy(
                src_ref=acc_ref.at[pl.ds(b * Sq, Sq), pl.ds(wing * CW, CW)],
                dst_ref=recv_ref.at[idx],
                send_sem=send_sems.at[idx],
                recv_sem=recv_sems.at[idx],
                device_id=(partner,),
                device_id_type=pl.DeviceIdType.MESH,
            )
            rdma.start()
            return rdma

        X = {}
        for b in range(B):
            xb = xv[b].astype(BF16)
            q = jnp.dot(xb, wq16, preferred_element_type=F32)
            k = jnp.dot(xb, wk16, preferred_element_type=F32)
            v = jnp.dot(xb, wv16, preferred_element_type=F32)
            q = q * cos + jnp.dot(q.astype(BF16), rot,
                                  preferred_element_type=F32) * sin
            k = k * cos + jnp.dot(k.astype(BF16), rot,
                                  preferred_element_type=F32) * sin
            q, k, v = q.astype(BF16), k.astype(BF16), v.astype(BF16)
            ctxs = []
            for h in range(HQ_LOCAL):
                cols = slice(h * Dh, (h + 1) * Dh)
                qh, kh, vh = q[:, cols], k[:, cols], v[:, cols]
                s = lax.dot_general(
                    qh, kh, (((1,), (1,)), ((), ())),
                    preferred_element_type=F32,
                ) * 0.125
                s = s - jnp.max(s, axis=-1, keepdims=True)
                w = jnp.exp(s)
                w = (w / jnp.sum(w, axis=-1, keepdims=True)).astype(BF16)
                ctxs.append(jnp.dot(w, vh, preferred_element_type=F32))
            ctx = jnp.concatenate(ctxs, axis=1).astype(BF16)
            acc_ref[pl.ds(b * Sq, Sq), :] = jnp.dot(
                ctx, wo16, preferred_element_type=F32).astype(BF16)
            X[(0, 0, b)] = exchange(0, 0, b, pa)
            X[(0, 1, b)] = exchange(0, 1, b, pb)

        for b in range(B):
            X[(0, 0, b)].wait()
            X[(0, 1, b)].wait()
            acc_ref[pl.ds(b * Sq, Sq), pl.ds(0, CW)] += recv_ref[0 + b]
            acc_ref[pl.ds(b * Sq, Sq), pl.ds(CW, CW)] += recv_ref[2 + b]
            X[(1, 0, b)] = exchange(1, 0, b, pb)
            X[(1, 1, b)] = exchange(1, 1, b, pa)

        stores = []
        for b in range(B):
            X[(1, 0, b)].wait()
            X[(1, 1, b)].wait()
            acc_ref[pl.ds(b * Sq, Sq), pl.ds(0, CW)] += recv_ref[4 + b]
            acc_ref[pl.ds(b * Sq, Sq), pl.ds(CW, CW)] += recv_ref[6 + b]
            st = pltpu.make_async_copy(
                acc_ref.at[pl.ds(b * Sq, Sq), :],
                out_hbm.at[b],
                store_sems.at[b],
            )
            st.start()
            stores.append(st)
        for st in stores:
            st.wait()

    hbm = pl.BlockSpec(memory_space=pltpu.MemorySpace.HBM)
    out3d = pl.pallas_call(
        body,
        out_shape=jax.ShapeDtypeStruct((B, Sq, D), BF16),
        in_specs=[hbm] * 8,
        out_specs=hbm,
        scratch_shapes=[
            pltpu.VMEM((B, Sq, D), F32),
            pltpu.VMEM((D, HD), F32),
            pltpu.VMEM((D, HD), F32),
            pltpu.VMEM((D, HD), F32),
            pltpu.VMEM((HD, D), F32),
            pltpu.VMEM((Sq, HD), F32),
            pltpu.VMEM((Sq, HD), F32),
            pltpu.VMEM((HD, HD), BF16),
            pltpu.VMEM((B * Sq, D), BF16),
            pltpu.VMEM((8, Sq, CW), BF16),
            pltpu.SemaphoreType.DMA((8,)),
            pltpu.SemaphoreType.DMA((2,)),
            pltpu.SemaphoreType.DMA((8,)),
            pltpu.SemaphoreType.DMA((8,)),
        ],
        compiler_params=pltpu.CompilerParams(collective_id=0),
    )(x, Wq, Wk, Wv, Wo, cos4, sin4, rot4)
    return out3d


# device time: 19427 ns/iter; 1.2788x vs baseline; 1.2788x over previous
import numpy as np
import jax
import jax.numpy as jnp
from jax import lax
from jax.experimental import pallas as pl
from jax.experimental.pallas import tpu as pltpu

N_DEV = 4
B, Sq, D = 2, 256, 768
HQ_LOCAL, Dh = 4, 64
HD = HQ_LOCAL * Dh
CW = D // 2

X_R, W3_R, WO_R, TRIG_R, PACK_R = 0, 512, 1280, 1536, 1792

F32 = jnp.float32
BF16 = jnp.bfloat16


def _rope_consts():
    inv = 1.0 / (10000.0 ** (np.arange(0, Dh, 2) / Dh))
    pos = np.arange(Sq)[:, None] * inv[None, :]
    cos = np.repeat(np.cos(pos), 2, axis=-1).astype(np.float32)
    sin = np.repeat(np.sin(pos), 2, axis=-1).astype(np.float32)
    rot = np.zeros((Dh, Dh), np.float32)
    for i in range(Dh // 2):
        rot[2 * i + 1, 2 * i] = -1.0
        rot[2 * i, 2 * i + 1] = 1.0
    cos4 = np.tile(cos, (1, HQ_LOCAL))
    sin4 = np.tile(sin, (1, HQ_LOCAL))
    rot4 = np.kron(np.eye(HQ_LOCAL, dtype=np.float32), rot)
    return jnp.asarray(cos4), jnp.asarray(sin4), jnp.asarray(rot4)


def kernel(x, Wq, Wk, Wv, Wo):
    cos4, sin4, rot4 = _rope_consts()
    pack = jnp.concatenate(
        [
            x.reshape(B * Sq, D),
            jnp.concatenate([Wq, Wk, Wv], axis=1),
            Wo,
            jnp.concatenate([cos4, sin4, rot4], axis=1),
        ],
        axis=0,
    ).astype(BF16)

    def body(pack_hbm, out_hbm, pk, acc_ref, recv_ref,
             load_sem, store_sems, send_sems, recv_sems):
        my = lax.axis_index("i")
        pa = 3 - my
        pb = my ^ 1

        load = pltpu.make_async_copy(pack_hbm, pk, load_sem)
        load.start()

        barrier_sem = pltpu.get_barrier_semaphore()
        for nbr in [pa, pb]:
            pl.semaphore_signal(
                barrier_sem, inc=1,
                device_id=(nbr,), device_id_type=pl.DeviceIdType.MESH,
            )
        load.wait()

        w3 = pk[W3_R:WO_R, :]
        wo = pk[WO_R:TRIG_R, :]
        cos = pk[TRIG_R:PACK_R, 0:HD].astype(F32)
        sin = pk[TRIG_R:PACK_R, HD:2 * HD].astype(F32)
        rot = pk[TRIG_R:PACK_R, 2 * HD:D]

        def exchange(step, wing, b, partner):
            idx = step * 4 + wing * 2 + b
            rdma = pltpu.make_async_remote_copy(
                src_ref=acc_ref.at[pl.ds(b * Sq, Sq), pl.ds(wing * CW, CW)],
                dst_ref=recv_ref.at[idx],
                send_sem=send_sems.at[idx],
                recv_sem=recv_sems.at[idx],
                device_id=(partner,),
                device_id_type=pl.DeviceIdType.MESH,
            )
            rdma.start()
            return rdma

        X = {}
        for b in range(B):
            xb = pk[X_R + b * Sq:X_R + (b + 1) * Sq, :]
            qkv = jnp.dot(xb, w3, preferred_element_type=F32)
            q = qkv[:, 0:HD]
            k = qkv[:, HD:2 * HD]
            v = qkv[:, 2 * HD:D].astype(BF16)
            q = q * cos + jnp.dot(q.astype(BF16), rot,
                                  preferred_element_type=F32) * sin
            k = k * cos + jnp.dot(k.astype(BF16), rot,
                                  preferred_element_type=F32) * sin
            q, k = q.astype(BF16), k.astype(BF16)
            ctxs = []
            for h in range(HQ_LOCAL):
                cols = slice(h * Dh, (h + 1) * Dh)
                qh, kh, vh = q[:, cols], k[:, cols], v[:, cols]
                s = lax.dot_general(
                    qh, kh, (((1,), (1,)), ((), ())),
                    preferred_element_type=F32,
                ) * 0.125
                s = s - jnp.max(s, axis=-1, keepdims=True)
                w = jnp.exp(s)
                w = (w / jnp.sum(w, axis=-1, keepdims=True)).astype(BF16)
                ctxs.append(jnp.dot(w, vh, preferred_element_type=F32))
            ctx = jnp.concatenate(ctxs, axis=1).astype(BF16)
            acc_ref[pl.ds(b * Sq, Sq), :] = jnp.dot(
                ctx, wo, preferred_element_type=F32).astype(BF16)
            if b == 0:
                pl.semaphore_wait(barrier_sem, 2)
            X[(0, 0, b)] = exchange(0, 0, b, pa)
            X[(0, 1, b)] = exchange(0, 1, b, pb)

        for b in range(B):
            X[(0, 0, b)].wait()
            X[(0, 1, b)].wait()
            acc_ref[pl.ds(b * Sq, Sq), pl.ds(0, CW)] += recv_ref[0 + b]
            acc_ref[pl.ds(b * Sq, Sq), pl.ds(CW, CW)] += recv_ref[2 + b]
            X[(1, 0, b)] = exchange(1, 0, b, pb)
            X[(1, 1, b)] = exchange(1, 1, b, pa)

        stores = []
        for b in range(B):
            X[(1, 0, b)].wait()
            X[(1, 1, b)].wait()
            acc_ref[pl.ds(b * Sq, Sq), pl.ds(0, CW)] += recv_ref[4 + b]
            acc_ref[pl.ds(b * Sq, Sq), pl.ds(CW, CW)] += recv_ref[6 + b]
            st = pltpu.make_async_copy(
                acc_ref.at[pl.ds(b * Sq, Sq), :],
                out_hbm.at[b],
                store_sems.at[b],
            )
            st.start()
            stores.append(st)
        for st in stores:
            st.wait()

    hbm = pl.BlockSpec(memory_space=pltpu.MemorySpace.HBM)
    out3d = pl.pallas_call(
        body,
        out_shape=jax.ShapeDtypeStruct((B, Sq, D), BF16),
        in_specs=[hbm],
        out_specs=hbm,
        scratch_shapes=[
            pltpu.VMEM((PACK_R, D), BF16),
            pltpu.VMEM((B * Sq, D), BF16),
            pltpu.VMEM((8, Sq, CW), BF16),
            pltpu.SemaphoreType.DMA,
            pltpu.SemaphoreType.DMA((2,)),
            pltpu.SemaphoreType.DMA((8,)),
            pltpu.SemaphoreType.DMA((8,)),
        ],
        compiler_params=pltpu.CompilerParams(collective_id=0),
    )(pack)
    return out3d
